# baseline (speedup 1.0000x reference)
import functools

import jax
import jax.numpy as jnp
from jax import lax
from jax.experimental import pallas as pl
from jax.experimental.pallas import tpu as pltpu
from jax.experimental.pallas import tpu_sc as plsc

NC, NS, L = 2, 16, 16
NW = NC * NS
ROWS = 64 * 32
ROW_LEN = 32768
NIDX = 1024
ROWS_PER_W = ROWS // NW
NPAIR = ROWS_PER_W // 2

_mesh = plsc.VectorSubcoreMesh(
    core_axis_name="c", subcore_axis_name="s", num_cores=NC, num_subcores=NS
)


@functools.partial(
    pl.kernel,
    out_type=jax.ShapeDtypeStruct((ROWS, NIDX), jnp.float32),
    mesh=_mesh,
    compiler_params=pltpu.CompilerParams(needs_layout_passes=False),
    scratch_types=[
        pltpu.VMEM((NIDX,), jnp.int32),
        pltpu.VMEM((NIDX,), jnp.int32),
        pltpu.VMEM((NIDX,), jnp.float32),
        pltpu.VMEM((NIDX,), jnp.float32),
        pltpu.SemaphoreType.DMA,
        pltpu.SemaphoreType.DMA,
        pltpu.SemaphoreType.DMA,
        pltpu.SemaphoreType.DMA,
        pltpu.SemaphoreType.DMA,
        pltpu.SemaphoreType.DMA,
    ],
)
def _sc_gather(t_hbm, i_hbm, o_hbm, idx0_v, idx1_v, out0_v, out1_v,
               isem0, isem1, gsem0, gsem1, osem0, osem1):
    wid = lax.axis_index("s") * NC + lax.axis_index("c")
    base = wid * ROWS_PER_W
    idxs_v = (idx0_v, idx1_v)
    outs_v = (out0_v, out1_v)
    isems = (isem0, isem1)
    gsems = (gsem0, gsem1)
    osems = (osem0, osem1)

    def start_idx(row, p):
        pltpu.async_copy(i_hbm.at[row], idxs_v[p], isems[p])

    def wait_idx(row, p):
        pltpu.make_async_copy(i_hbm.at[row], idxs_v[p], isems[p]).wait()

    def issue_gathers(row, p):
        fb = row * ROW_LEN
        for i in range(NIDX // L):
            sl = pl.ds(i * L, L)
            iv = idxs_v[p][sl] + fb
            pltpu.async_copy(t_hbm.at[iv], outs_v[p].at[sl], gsems[p])

    def drain_gathers(p):
        pltpu.make_async_copy(t_hbm.at[pl.ds(0, NIDX)], outs_v[p],
                              gsems[p]).wait()

    def start_out(row, p):
        pltpu.async_copy(outs_v[p], o_hbm.at[row], osems[p])

    def wait_out(p):
        pltpu.make_async_copy(outs_v[p], o_hbm.at[base], osems[p]).wait()

    start_idx(base, 0)
    wait_idx(base, 0)
    start_idx(base + 1, 1)
    issue_gathers(base, 0)

    def pair(g, _):
        r0 = base + 2 * g
        wait_idx(r0 + 1, 1)
        @pl.when(g > 0)
        def _():
            wait_out(1)
        issue_gathers(r0 + 1, 1)
        @pl.when(g < NPAIR - 1)
        def _():
            start_idx(r0 + 2, 0)
        drain_gathers(0)
        start_out(r0, 0)
        @pl.when(g < NPAIR - 1)
        def _():
            wait_idx(r0 + 2, 0)
            wait_out(0)
            issue_gathers(r0 + 2, 0)
            start_idx(r0 + 3, 1)
        drain_gathers(1)
        start_out(r0 + 1, 1)
        return 0

    lax.fori_loop(0, NPAIR, pair, 0)
    wait_out(0)
    wait_out(1)


def kernel(tensor, indices):
    t = tensor.reshape(ROWS * ROW_LEN)
    ix = indices.reshape(ROWS, NIDX)
    out = _sc_gather(t, ix)
    return out.reshape(indices.shape)
